# Initial kernel scaffold; baseline (speedup 1.0000x reference)
import jax
import jax.numpy as jnp
from jax.experimental import pallas as pl


def kernel(x, edge_index, edge_attr, u, batch, W1, b1, W2, b2, W3, b3):
    raise NotImplementedError("write your pallas kernel here")


# speedup vs baseline: 3.4329x; 3.4329x over previous
import functools

import jax
import jax.numpy as jnp
from jax import lax
from jax.experimental import pallas as pl
from jax.experimental.pallas import tpu as pltpu
from jax.experimental.pallas import tpu_sc as plsc

N = 100000
D = 322
G = 256
HID = 256
LAT = 64
NC, NS, L = 2, 16, 16
NW = NC * NS
SPG = G // NW
NCH = (D + L - 1) // L
DP = NCH * L
IDC = 2560
NPAD = 41 * IDC
R = 128
RS = R - 8


def _off(c):
    return c * L if c < NCH - 1 else D - L


def _vsum_i32(vec):
    tot = vec[0]
    for j in range(1, L):
        tot = tot + vec[j]
    return tot


def _sc_pool_body(x_hbm, batch_hbm, out_hbm, ids_v, row_v, acc_s, acc_mx,
                  acc_mn, stg):
    wid = lax.axis_index("s") * NC + lax.axis_index("c")
    lo = (wid * SPG).astype(jnp.int32)

    zero = jnp.zeros((L,), jnp.int32)

    def ch_a(i, car):
        pltpu.sync_copy(batch_hbm.at[pl.ds(i * IDC, IDC)], ids_v)

        def in_a(j, car2):
            lo_c, hi_c = car2
            v = ids_v[pl.ds(j * L, L)]
            return (lo_c + jnp.where(v < lo, 1, 0).astype(jnp.int32),
                    hi_c + jnp.where(v < lo + SPG, 1, 0).astype(jnp.int32))

        return lax.fori_loop(0, IDC // L, in_a, car)

    lo_c, hi_c = lax.fori_loop(0, (N + IDC - 1) // IDC, ch_a, (zero, zero))
    start = _vsum_i32(lo_c)
    end = _vsum_i32(hi_c)

    a0 = pl.multiple_of((start // 128) * 128, 128)

    def ch_b(i, car):
        pltpu.sync_copy(batch_hbm.at[pl.ds(a0 + i * IDC, IDC)], ids_v)

        def in_b(j, car2):
            v = ids_v[pl.ds(j * L, L)]
            return tuple(c + jnp.where(v == lo + k, 1, 0).astype(jnp.int32)
                         for k, c in enumerate(car2))

        return lax.fori_loop(0, IDC // L, in_b, car)

    nch2 = (end - a0 + IDC - 1) // IDC
    zeros8 = tuple(jnp.zeros((L,), jnp.int32) for _ in range(SPG))
    cnt_vs = lax.fori_loop(0, nch2, ch_b, zeros8)
    cnts = [_vsum_i32(c) for c in cnt_vs]

    neg_inf = jnp.full((L,), -jnp.inf, jnp.float32)
    pos_inf = jnp.full((L,), jnp.inf, jnp.float32)
    gs = start
    for k in range(SPG):
        cnt = cnts[k]
        ge = gs + cnt
        for c in range(NCH):
            asl = pl.ds(c * L, L)
            acc_s[asl] = jnp.zeros((L,), jnp.float32)
            acc_mx[asl] = neg_inf
            acc_mn[asl] = pos_inf

        def chunk_body(i, _, gs=gs, ge=ge):
            pos = gs + i * RS
            m = jnp.minimum(RS, ge - pos)
            cs = pl.multiple_of(jnp.minimum((pos // 8) * 8, N - R), 8)
            o0 = pos - cs
            pltpu.sync_copy(x_hbm.at[pl.ds(cs, R)], row_v)
            for c in range(NCH):
                oc = _off(c)
                asl = pl.ds(c * L, L)

                def row_body(r, car2, oc=oc):
                    s, mx, mn = car2
                    v = row_v[o0 + r, pl.ds(oc, L)]
                    return (s + v, jnp.maximum(mx, v), jnp.minimum(mn, v))

                s, mx, mn = lax.fori_loop(
                    0, m, row_body,
                    (jnp.zeros((L,), jnp.float32), neg_inf, pos_inf))
                acc_s[asl] = acc_s[asl] + s
                acc_mx[asl] = jnp.maximum(acc_mx[asl], mx)
                acc_mn[asl] = jnp.minimum(acc_mn[asl], mn)
            return 0

        nck = (cnt + RS - 1) // RS
        lax.fori_loop(0, nck, chunk_body, 0)

        denom = jnp.maximum(cnt.astype(jnp.float32), 1.0)
        for c in range(NCH):
            asl = pl.ds(c * L, L)
            stg[0, k, asl] = acc_s[asl] / denom
            stg[1, k, asl] = acc_mx[asl]
            stg[2, k, asl] = acc_mn[asl]
        gs = ge

    lo8 = pl.multiple_of(lo, 8)
    for t in range(3):
        pltpu.sync_copy(stg.at[t], out_hbm.at[t, pl.ds(lo8, SPG)])


def _sc_pool(x, batch_pad):
    mesh = plsc.VectorSubcoreMesh(core_axis_name="c", subcore_axis_name="s",
                                  num_cores=NC, num_subcores=NS)
    return pl.kernel(
        _sc_pool_body,
        out_type=jax.ShapeDtypeStruct((3, G, DP), jnp.float32),
        mesh=mesh,
        scratch_types=[
            pltpu.VMEM((IDC,), jnp.int32),
            pltpu.VMEM((R, D), jnp.float32),
            pltpu.VMEM((DP,), jnp.float32),
            pltpu.VMEM((DP,), jnp.float32),
            pltpu.VMEM((DP,), jnp.float32),
            pltpu.VMEM((3, SPG, DP), jnp.float32),
        ],
    )(x, batch_pad)


def _mlp_body(a_ref, w1_ref, b1_ref, w2_ref, b2_ref, w3_ref, b3_ref, o_ref):
    dot = functools.partial(jnp.dot, preferred_element_type=jnp.float32,
                            precision=lax.Precision.DEFAULT)
    h = jnp.maximum(dot(a_ref[...], w1_ref[...]) + b1_ref[...], 0.0)
    h = jnp.maximum(dot(h, w2_ref[...]) + b2_ref[...], 0.0)
    o_ref[...] = dot(h, w3_ref[...]) + b3_ref[...]


def _mlp(a, W1, b1, W2, b2, W3, b3):
    return pl.pallas_call(
        _mlp_body,
        out_shape=jax.ShapeDtypeStruct((a.shape[0], LAT), jnp.float32),
    )(a, W1, b1, W2, b2, W3, b3)


def kernel(x, edge_index, edge_attr, u, batch, W1, b1, W2, b2, W3, b3):
    batch_pad = jnp.concatenate(
        [batch, jnp.full((NPAD - N,), G, jnp.int32)])
    pooled = _sc_pool(x, batch_pad)
    parts = [u]
    for t in range(3):
        parts.append(pooled[t, :, :(NCH - 1) * L])
        parts.append(pooled[t, :, DP - 2:DP])
    mlp_in = jnp.concatenate(parts, axis=1)
    return _mlp(mlp_in, W1, b1.reshape(1, -1), W2, b2.reshape(1, -1),
                W3, b3.reshape(1, -1))
